# baseline (speedup 1.0000x reference)
import jax
import jax.numpy as jnp
from jax import lax
from jax.experimental import pallas as pl
from jax.experimental.pallas import tpu as pltpu
from jax.experimental.pallas import tpu_sc as plsc

B = 16384
NE = 1000000
NR = 1000
D = 64
W = 2 * D
L = 16
NC = 2
NS = 16
NW = NC * NS
BPW = B // NW
CHUNK = 128
NCHUNK = BPW // CHUNK

EB = 49152
GRID = (NE + EB - 1) // EB
FR = GRID * (EB // 2)


def _format_tc(x_ref, o_ref):
    for m in range(EB // 256):
        xa = x_ref[:, m * 256:m * 256 + 128]
        xb = x_ref[:, m * 256 + 128:m * 256 + 256]
        x2 = jnp.concatenate([xa, xb], axis=0)
        o_ref[m * 128:(m + 1) * 128, :] = jnp.transpose(x2)


def _score_sc(ph_hbm, pr_hbm, pt_hbm, nh_hbm, nt_hbm, ent_hbm, rel_hbm,
              pos_hbm, neg_hbm,
              phv, prv, ptv, nhv, ntv,
              phh, pth, nhh, nth,
              ph_rows, pt_rows, nh_rows, nt_rows, r_rows,
              pos_v, neg_v, sem):
    wid = lax.axis_index("s") * NC + lax.axis_index("c")
    lane = lax.iota(jnp.int32, L)

    def chunk_body(c, chunk_carry):
        base = wid * BPW + c * CHUNK
        sl = pl.ds(base, CHUNK)
        pltpu.sync_copy(ph_hbm.at[sl], phv)
        pltpu.sync_copy(pr_hbm.at[sl], prv)
        pltpu.sync_copy(pt_hbm.at[sl], ptv)
        pltpu.sync_copy(nh_hbm.at[sl], nhv)
        pltpu.sync_copy(nt_hbm.at[sl], ntv)

        def rowmap(i, carry):
            ds16 = pl.ds(i * L, L)
            phh[ds16] = ((phv[ds16] >> 8) << 7) + (phv[ds16] & 127)
            pth[ds16] = ((ptv[ds16] >> 8) << 7) + (ptv[ds16] & 127)
            nhh[ds16] = ((nhv[ds16] >> 8) << 7) + (nhv[ds16] & 127)
            nth[ds16] = ((ntv[ds16] >> 8) << 7) + (ntv[ds16] & 127)
            return carry

        lax.fori_loop(0, CHUNK // L, rowmap, 0)

        g1 = pltpu.async_copy(ent_hbm.at[phh], ph_rows, sem)
        g2 = pltpu.async_copy(ent_hbm.at[pth], pt_rows, sem)
        g3 = pltpu.async_copy(ent_hbm.at[nhh], nh_rows, sem)
        g4 = pltpu.async_copy(ent_hbm.at[nth], nt_rows, sem)
        g5 = pltpu.async_copy(rel_hbm.at[prv], r_rows, sem)
        g1.wait(); g2.wait(); g3.wait(); g4.wait(); g5.wait()

        def body(g, carry):
            ds16 = pl.ds(g * L, L)
            rowidx = g * L + lane
            phi = phv[ds16]
            pti = ptv[ds16]
            nhi = nhv[ds16]
            nti = ntv[ds16]
            phb = ((phi >> 7) & 1) * D
            ptb = ((pti >> 7) & 1) * D
            nhb = ((nhi >> 7) & 1) * D
            ntb = ((nti >> 7) & 1) * D
            pacc = jnp.zeros((L,), jnp.float32)
            nacc = jnp.zeros((L,), jnp.float32)
            for d in range(D):
                rcol = (lane + d) & (D - 1)
                r = plsc.load_gather(r_rows, [rowidx, rcol])
                ph = plsc.load_gather(ph_rows, [rowidx, rcol + phb])
                pt = plsc.load_gather(pt_rows, [rowidx, rcol + ptb])
                nh = plsc.load_gather(nh_rows, [rowidx, rcol + nhb])
                nt = plsc.load_gather(nt_rows, [rowidx, rcol + ntb])
                pacc = pacc + jnp.abs(ph + r - pt)
                nacc = nacc + jnp.abs(nh + r - nt)
            pos_v[pl.ds(g * L, L)] = pacc
            neg_v[pl.ds(g * L, L)] = nacc
            return carry

        lax.fori_loop(0, CHUNK // L, body, 0)
        pltpu.sync_copy(pos_v, pos_hbm.at[sl])
        pltpu.sync_copy(neg_v, neg_hbm.at[sl])
        return chunk_carry

    lax.fori_loop(0, NCHUNK, chunk_body, 0)


@jax.jit
def kernel(pos_samples, neg_samples, entity_table, relation_table):
    ph = pos_samples[:, 0].astype(jnp.int32)
    pr = pos_samples[:, 1].astype(jnp.int32)
    pt = pos_samples[:, 2].astype(jnp.int32)
    nh = neg_samples[:, 0].astype(jnp.int32)
    nt = neg_samples[:, 2].astype(jnp.int32)
    entT = entity_table.T
    relp = jnp.pad(relation_table, ((0, 0), (0, W - D)))

    fmt = pl.pallas_call(
        _format_tc,
        grid=(GRID,),
        in_specs=[pl.BlockSpec((D, EB), lambda j: (0, j))],
        out_specs=pl.BlockSpec((EB // 2, W), lambda j: (j, 0)),
        out_shape=jax.ShapeDtypeStruct((FR, W), jnp.float32),
        compiler_params=pltpu.CompilerParams(
            dimension_semantics=("arbitrary",)),
    )(entT)

    mesh = plsc.VectorSubcoreMesh(core_axis_name="c", subcore_axis_name="s")
    params = pltpu.CompilerParams(
        needs_layout_passes=False, use_tc_tiling_on_sc=True)

    score = pl.kernel(
        _score_sc,
        out_type=(
            jax.ShapeDtypeStruct((B,), jnp.float32),
            jax.ShapeDtypeStruct((B,), jnp.float32),
        ),
        mesh=mesh,
        compiler_params=params,
        scratch_types=[
            pltpu.VMEM((CHUNK,), jnp.int32),
            pltpu.VMEM((CHUNK,), jnp.int32),
            pltpu.VMEM((CHUNK,), jnp.int32),
            pltpu.VMEM((CHUNK,), jnp.int32),
            pltpu.VMEM((CHUNK,), jnp.int32),
            pltpu.VMEM((CHUNK,), jnp.int32),
            pltpu.VMEM((CHUNK,), jnp.int32),
            pltpu.VMEM((CHUNK,), jnp.int32),
            pltpu.VMEM((CHUNK,), jnp.int32),
            pltpu.VMEM((CHUNK, W), jnp.float32),
            pltpu.VMEM((CHUNK, W), jnp.float32),
            pltpu.VMEM((CHUNK, W), jnp.float32),
            pltpu.VMEM((CHUNK, W), jnp.float32),
            pltpu.VMEM((CHUNK, W), jnp.float32),
            pltpu.VMEM((CHUNK,), jnp.float32),
            pltpu.VMEM((CHUNK,), jnp.float32),
            pltpu.SemaphoreType.DMA,
        ],
    )
    return score(ph, pr, pt, nh, nt, fmt, relp)
